# baseline (speedup 1.0000x reference)
import functools

import jax
import jax.numpy as jnp
from jax import lax
from jax.experimental import pallas as pl
from jax.experimental.pallas import tpu as pltpu
from jax.experimental.pallas import tpu_sc as plsc


def _iota16():
    return lax.iota(jnp.int32, 16)


def _splat16(c):
    return jnp.full((16,), c, jnp.int32)


def _transpose_block(src, dst, n_l):
    it = _iota16()
    rows0 = it
    rows1 = it + 16
    for l in range(n_l):
        col = _splat16(l)
        dst[l, pl.ds(0, 16)] = plsc.load_gather(src, [rows0, col])
        dst[l, pl.ds(16, 16)] = plsc.load_gather(src, [rows1, col])


_FULL = 7812
_MAIN = 244


@functools.cache
def _make_rearrange(V, D):
    assert V == 1000000 and D == 32
    info = plsc.get_sparse_core_info()
    NC, NS = info.num_cores, info.num_subcores
    mesh = plsc.VectorSubcoreMesh(core_axis_name="c", subcore_axis_name="s")

    @functools.partial(
        pl.kernel,
        mesh=mesh,
        compiler_params=pltpu.CompilerParams(use_tc_tiling_on_sc=True),
        out_type=jax.ShapeDtypeStruct((_FULL * 32, 128), jnp.float32),
        scratch_types=[
            pltpu.VMEM((32, 128), jnp.float32),
            pltpu.VMEM((32, 128), jnp.float32),
            pltpu.SemaphoreType.DMA,
            pltpu.SemaphoreType.DMA,
            pltpu.SemaphoreType.DMA,
            pltpu.SemaphoreType.DMA,
        ],
    )
    def k(tt_hbm, out_hbm, buf0, buf1, si0, si1, so0, so1):
        wid = lax.axis_index("s") * NC + lax.axis_index("c")
        base_t = wid * _MAIN
        bufs = (buf0, buf1)
        sis = (si0, si1)
        sos = (so0, so1)

        def fire_in(t, b):
            pltpu.make_async_copy(
                tt_hbm.at[:, pl.ds(t * 128, 128)], bufs[b], sis[b]
            ).start()

        def wait_in(b):
            pltpu.make_async_copy(
                tt_hbm.at[:, pl.ds(0, 128)], bufs[b], sis[b]
            ).wait()

        def fire_out(t, b):
            pltpu.make_async_copy(
                bufs[b], out_hbm.at[pl.ds(t * 32, 32), :], sos[b]
            ).start()

        def wait_out(b):
            pltpu.make_async_copy(
                bufs[b], out_hbm.at[pl.ds(0, 32), :], sos[b]
            ).wait()

        fire_in(base_t, 0)
        fire_in(base_t + 1, 1)

        def body2(g, carry):
            for b in range(2):
                t = base_t + 2 * g + b
                wait_in(b)
                fire_out(t, b)
                wait_out(b)

                @pl.when(g < _MAIN // 2 - 1)
                def _():
                    fire_in(t + 2, b)

            return carry

        lax.fori_loop(0, _MAIN // 2, body2, 0)

        @pl.when(wid < 4)
        def _():
            t = 32 * _MAIN + wid
            pltpu.sync_copy(tt_hbm.at[:, pl.ds(t * 128, 128)], buf0)
            pltpu.sync_copy(buf0, out_hbm.at[pl.ds(t * 32, 32), :])

    return k


@functools.cache
def _make_transpose(V, D):
    assert V == 1000000 and D == 32
    info = plsc.get_sparse_core_info()
    NC, NS = info.num_cores, info.num_subcores
    mesh = plsc.VectorSubcoreMesh(core_axis_name="c", subcore_axis_name="s")

    @functools.partial(
        pl.kernel,
        mesh=mesh,
        compiler_params=pltpu.CompilerParams(
            use_tc_tiling_on_sc=False, needs_layout_passes=False),
        out_type=jax.ShapeDtypeStruct((V, D), jnp.float32),
        scratch_types=[
            pltpu.VMEM((32, 128), jnp.float32),
            pltpu.VMEM((32, 128), jnp.float32),
            pltpu.VMEM((128, 32), jnp.float32),
            pltpu.VMEM((128, 32), jnp.float32),
            pltpu.SemaphoreType.DMA,
            pltpu.SemaphoreType.DMA,
            pltpu.SemaphoreType.DMA,
            pltpu.SemaphoreType.DMA,
        ],
    )
    def k(in_hbm, tail_hbm, out_hbm, buf0, buf1, tr0, tr1, si0, si1, so0, so1):
        wid = lax.axis_index("s") * NC + lax.axis_index("c")
        base_t = wid * _MAIN
        bufs = (buf0, buf1)
        trs = (tr0, tr1)
        sis = (si0, si1)
        sos = (so0, so1)

        def fire_in(t, b):
            pltpu.make_async_copy(
                in_hbm.at[pl.ds(t * 32, 32), :], bufs[b], sis[b]
            ).start()

        def wait_in(b):
            pltpu.make_async_copy(
                in_hbm.at[pl.ds(0, 32), :], bufs[b], sis[b]
            ).wait()

        def fire_out(t, b):
            pltpu.make_async_copy(
                trs[b], out_hbm.at[pl.ds(t * 128, 128), :], sos[b]
            ).start()

        def wait_out(b):
            pltpu.make_async_copy(
                trs[b], out_hbm.at[pl.ds(0, 128), :], sos[b]
            ).wait()

        fire_in(base_t, 0)
        fire_in(base_t + 1, 1)

        def body(g, carry):
            for b in range(2):
                t = base_t + 2 * g + b
                wait_in(b)

                @pl.when(g > 0)
                def _():
                    wait_out(b)

                _transpose_block(bufs[b], trs[b], 128)

                @pl.when(g < _MAIN // 2 - 1)
                def _():
                    fire_in(t + 2, b)

                fire_out(t, b)
            return carry

        lax.fori_loop(0, _MAIN // 2, body, 0)
        wait_out(0)
        wait_out(1)

        @pl.when(wid < 4)
        def _():
            t = 32 * _MAIN + wid
            pltpu.sync_copy(in_hbm.at[pl.ds(t * 32, 32), :], buf0)
            _transpose_block(buf0, tr0, 128)
            pltpu.sync_copy(tr0, out_hbm.at[pl.ds(t * 128, 128), :])

        @pl.when(wid == 4)
        def _():
            pltpu.sync_copy(tail_hbm, tr1.at[pl.ds(0, 64), :])
            pltpu.sync_copy(
                tr1.at[pl.ds(0, 64), :],
                out_hbm.at[pl.ds(_FULL * 128, 64), :],
            )

    return k


@functools.cache
def _make_gather(V, D, NB, S):
    assert D == 32 and S == 50 and NB % (128 * 32) == 0
    info = plsc.get_sparse_core_info()
    NC, NS = info.num_cores, info.num_subcores
    NW = NC * NS
    NT = NB // 128
    TPW = NT // NW
    KB = 5
    mesh = plsc.VectorSubcoreMesh(core_axis_name="c", subcore_axis_name="s")

    @functools.partial(
        pl.kernel,
        mesh=mesh,
        compiler_params=pltpu.CompilerParams(
            use_tc_tiling_on_sc=False, needs_layout_passes=False),
        out_type=jax.ShapeDtypeStruct((S, 4, NT, 8, 128), jnp.float32),
        scratch_types=[
            pltpu.VMEM((128 * S,), jnp.int32),
            pltpu.VMEM((S, 128), jnp.int32),
            pltpu.VMEM((KB, 128, D), jnp.float32),
            pltpu.VMEM((KB, 4, 8, 128), jnp.float32),
            pltpu.SemaphoreType.DMA,
            pltpu.SemaphoreType.DMA,
            pltpu.SemaphoreType.DMA,
            pltpu.SemaphoreType.DMA,
            pltpu.SemaphoreType.DMA,
            pltpu.SemaphoreType.DMA,
            pltpu.SemaphoreType.DMA,
            pltpu.SemaphoreType.DMA,
            pltpu.SemaphoreType.DMA,
            pltpu.SemaphoreType.DMA,
        ],
    )
    def k(table_hbm, idx_hbm, out_hbm, idxbuf, ind, rowb, trb, *sems):
        sg = sems[:KB]
        sw = sems[KB:]
        wid = lax.axis_index("s") * NC + lax.axis_index("c")
        it = _iota16()
        i50 = it * S

        def fire_gather(j, b):
            pltpu.make_async_copy(
                table_hbm.at[ind.at[j]], rowb.at[b], sg[b]
            ).start()

        def wait_gather(b):
            pltpu.make_async_copy(
                table_hbm.at[pl.ds(0, 128)], rowb.at[b], sg[b]
            ).wait()

        def fire_writes(j, btg, b):
            for dg in range(4):
                pltpu.make_async_copy(
                    trb.at[b, dg], out_hbm.at[j, dg, btg], sw[b]
                ).start()

        def wait_writes(b):
            for dg in range(4):
                pltpu.make_async_copy(
                    trb.at[b, dg], out_hbm.at[0, dg, 0], sw[b]
                ).wait()

        def bt_body(btl, carry):
            btg = wid * TPW + btl
            pltpu.sync_copy(idx_hbm.at[pl.ds(btg * 128 * S, 128 * S)], idxbuf)
            for j in range(S):
                for h in range(8):
                    v = plsc.load_gather(idxbuf, [i50 + (h * 16 * S + j)])
                    ind[j, pl.ds(h * 16, 16)] = v
            for b in range(KB):
                fire_gather(b, b)

            def jg_body(g, c2):
                j0 = g * KB
                for b in range(KB):
                    j = j0 + b
                    wait_gather(b)

                    @pl.when(jnp.logical_or(g > 0, btl > 0))
                    def _():
                        wait_writes(b)

                    src = rowb.at[b]
                    dst = trb.at[b]
                    for dd in range(D):
                        col = _splat16(dd)
                        for l0 in range(0, 128, 16):
                            v = plsc.load_gather(src, [it + l0, col])
                            dst[dd // 8, dd % 8, pl.ds(l0, 16)] = v
                    fire_writes(j, btg, b)

                    @pl.when(g < S // KB - 1)
                    def _():
                        fire_gather(j + KB, b)

                return c2

            lax.fori_loop(0, S // KB, jg_body, 0)

            return carry

        lax.fori_loop(0, TPW, bt_body, 0)
        for b in range(KB):
            wait_writes(b)

    return k


def kernel(emojis, table):
    NB, S = emojis.shape
    V, D = table.shape
    idx = emojis.reshape(-1)
    table_t = table.T
    blocks = _make_rearrange(V, D)(table_t)
    tail = table[_FULL * 128 :, :]
    table_lin = _make_transpose(V, D)(blocks, tail)
    out5 = _make_gather(V, D, NB, S)(table_lin, idx)
    out = out5.transpose(2, 4, 0, 1, 3).reshape(NB, S, D)
    return out


# speedup vs baseline: 1.8606x; 1.8606x over previous
import functools

import jax
import jax.numpy as jnp
from jax import lax
from jax.experimental import pallas as pl
from jax.experimental.pallas import tpu as pltpu
from jax.experimental.pallas import tpu_sc as plsc


def _iota16():
    return lax.iota(jnp.int32, 16)


def _splat16(c):
    return jnp.full((16,), c, jnp.int32)


def _transpose_block(src, dst, n_l):
    it = _iota16()
    rows0 = it
    rows1 = it + 16

    @plsc.parallel_loop(0, n_l, unroll=8)
    def _(l):
        col = jnp.broadcast_to(l, (16,))
        dst[l, pl.ds(0, 16)] = plsc.load_gather(src, [rows0, col])
        dst[l, pl.ds(16, 16)] = plsc.load_gather(src, [rows1, col])


_FULL = 7812
_MAIN = 244


@functools.cache
def _make_rearrange(V, D):
    assert V == 1000000 and D == 32
    info = plsc.get_sparse_core_info()
    NC, NS = info.num_cores, info.num_subcores
    mesh = plsc.VectorSubcoreMesh(core_axis_name="c", subcore_axis_name="s")

    @functools.partial(
        pl.kernel,
        mesh=mesh,
        compiler_params=pltpu.CompilerParams(use_tc_tiling_on_sc=True),
        out_type=jax.ShapeDtypeStruct((_FULL * 32, 128), jnp.float32),
        scratch_types=[
            pltpu.VMEM((32, 128), jnp.float32),
            pltpu.VMEM((32, 128), jnp.float32),
            pltpu.SemaphoreType.DMA,
            pltpu.SemaphoreType.DMA,
            pltpu.SemaphoreType.DMA,
            pltpu.SemaphoreType.DMA,
        ],
    )
    def k(tt_hbm, out_hbm, buf0, buf1, si0, si1, so0, so1):
        wid = lax.axis_index("s") * NC + lax.axis_index("c")
        base_t = wid * _MAIN
        bufs = (buf0, buf1)
        sis = (si0, si1)
        sos = (so0, so1)

        def fire_in(t, b):
            pltpu.make_async_copy(
                tt_hbm.at[:, pl.ds(t * 128, 128)], bufs[b], sis[b]
            ).start()

        def wait_in(b):
            pltpu.make_async_copy(
                tt_hbm.at[:, pl.ds(0, 128)], bufs[b], sis[b]
            ).wait()

        def fire_out(t, b):
            pltpu.make_async_copy(
                bufs[b], out_hbm.at[pl.ds(t * 32, 32), :], sos[b]
            ).start()

        def wait_out(b):
            pltpu.make_async_copy(
                bufs[b], out_hbm.at[pl.ds(0, 32), :], sos[b]
            ).wait()

        fire_in(base_t, 0)
        fire_in(base_t + 1, 1)

        def body2(g, carry):
            for b in range(2):
                t = base_t + 2 * g + b
                wait_in(b)
                fire_out(t, b)
                wait_out(b)

                @pl.when(g < _MAIN // 2 - 1)
                def _():
                    fire_in(t + 2, b)

            return carry

        lax.fori_loop(0, _MAIN // 2, body2, 0)

        @pl.when(wid < 4)
        def _():
            t = 32 * _MAIN + wid
            pltpu.sync_copy(tt_hbm.at[:, pl.ds(t * 128, 128)], buf0)
            pltpu.sync_copy(buf0, out_hbm.at[pl.ds(t * 32, 32), :])

    return k


@functools.cache
def _make_transpose(V, D):
    assert V == 1000000 and D == 32
    info = plsc.get_sparse_core_info()
    NC, NS = info.num_cores, info.num_subcores
    mesh = plsc.VectorSubcoreMesh(core_axis_name="c", subcore_axis_name="s")

    @functools.partial(
        pl.kernel,
        mesh=mesh,
        compiler_params=pltpu.CompilerParams(
            use_tc_tiling_on_sc=False, needs_layout_passes=False),
        out_type=jax.ShapeDtypeStruct((V, D), jnp.float32),
        scratch_types=[
            pltpu.VMEM((32, 128), jnp.float32),
            pltpu.VMEM((32, 128), jnp.float32),
            pltpu.VMEM((128, 32), jnp.float32),
            pltpu.VMEM((128, 32), jnp.float32),
            pltpu.SemaphoreType.DMA,
            pltpu.SemaphoreType.DMA,
            pltpu.SemaphoreType.DMA,
            pltpu.SemaphoreType.DMA,
        ],
    )
    def k(in_hbm, tail_hbm, out_hbm, buf0, buf1, tr0, tr1, si0, si1, so0, so1):
        wid = lax.axis_index("s") * NC + lax.axis_index("c")
        base_t = wid * _MAIN
        bufs = (buf0, buf1)
        trs = (tr0, tr1)
        sis = (si0, si1)
        sos = (so0, so1)

        def fire_in(t, b):
            pltpu.make_async_copy(
                in_hbm.at[pl.ds(t * 32, 32), :], bufs[b], sis[b]
            ).start()

        def wait_in(b):
            pltpu.make_async_copy(
                in_hbm.at[pl.ds(0, 32), :], bufs[b], sis[b]
            ).wait()

        def fire_out(t, b):
            pltpu.make_async_copy(
                trs[b], out_hbm.at[pl.ds(t * 128, 128), :], sos[b]
            ).start()

        def wait_out(b):
            pltpu.make_async_copy(
                trs[b], out_hbm.at[pl.ds(0, 128), :], sos[b]
            ).wait()

        fire_in(base_t, 0)
        fire_in(base_t + 1, 1)

        def body(g, carry):
            for b in range(2):
                t = base_t + 2 * g + b
                wait_in(b)

                @pl.when(g > 0)
                def _():
                    wait_out(b)

                _transpose_block(bufs[b], trs[b], 128)

                @pl.when(g < _MAIN // 2 - 1)
                def _():
                    fire_in(t + 2, b)

                fire_out(t, b)
            return carry

        lax.fori_loop(0, _MAIN // 2, body, 0)
        wait_out(0)
        wait_out(1)

        @pl.when(wid < 4)
        def _():
            t = 32 * _MAIN + wid
            pltpu.sync_copy(in_hbm.at[pl.ds(t * 32, 32), :], buf0)
            _transpose_block(buf0, tr0, 128)
            pltpu.sync_copy(tr0, out_hbm.at[pl.ds(t * 128, 128), :])

        @pl.when(wid == 4)
        def _():
            pltpu.sync_copy(tail_hbm, tr1.at[pl.ds(0, 64), :])
            pltpu.sync_copy(
                tr1.at[pl.ds(0, 64), :],
                out_hbm.at[pl.ds(_FULL * 128, 64), :],
            )

    return k


@functools.cache
def _make_gather(V, D, NB, S):
    assert D == 32 and S == 50 and NB % (128 * 32) == 0
    info = plsc.get_sparse_core_info()
    NC, NS = info.num_cores, info.num_subcores
    NW = NC * NS
    NT = NB // 128
    TPW = NT // NW
    KB = 5
    mesh = plsc.VectorSubcoreMesh(core_axis_name="c", subcore_axis_name="s")

    @functools.partial(
        pl.kernel,
        mesh=mesh,
        compiler_params=pltpu.CompilerParams(
            use_tc_tiling_on_sc=False, needs_layout_passes=False),
        out_type=jax.ShapeDtypeStruct((S, 4, NT, 8, 128), jnp.float32),
        scratch_types=[
            pltpu.VMEM((128 * S,), jnp.int32),
            pltpu.VMEM((S, 128), jnp.int32),
            pltpu.VMEM((KB, 128, D), jnp.float32),
            pltpu.VMEM((KB, 4, 8, 128), jnp.float32),
            pltpu.SemaphoreType.DMA,
            pltpu.SemaphoreType.DMA,
            pltpu.SemaphoreType.DMA,
            pltpu.SemaphoreType.DMA,
            pltpu.SemaphoreType.DMA,
            pltpu.SemaphoreType.DMA,
            pltpu.SemaphoreType.DMA,
            pltpu.SemaphoreType.DMA,
            pltpu.SemaphoreType.DMA,
            pltpu.SemaphoreType.DMA,
        ],
    )
    def k(table_hbm, idx_hbm, out_hbm, idxbuf, ind, rowb, trb, *sems):
        sg = sems[:KB]
        sw = sems[KB:]
        wid = lax.axis_index("s") * NC + lax.axis_index("c")
        it = _iota16()
        i50 = it * S

        def fire_gather(j, b):
            pltpu.make_async_copy(
                table_hbm.at[ind.at[j]], rowb.at[b], sg[b]
            ).start()

        def wait_gather(b):
            pltpu.make_async_copy(
                table_hbm.at[pl.ds(0, 128)], rowb.at[b], sg[b]
            ).wait()

        def fire_writes(j, btg, b):
            for dg in range(4):
                pltpu.make_async_copy(
                    trb.at[b, dg], out_hbm.at[j, dg, btg], sw[b]
                ).start()

        def wait_writes(b):
            for dg in range(4):
                pltpu.make_async_copy(
                    trb.at[b, dg], out_hbm.at[0, dg, 0], sw[b]
                ).wait()

        def bt_body(btl, carry):
            btg = wid * TPW + btl
            pltpu.sync_copy(idx_hbm.at[pl.ds(btg * 128 * S, 128 * S)], idxbuf)

            @plsc.parallel_loop(0, S, unroll=5)
            def _(j):
                for h in range(8):
                    v = plsc.load_gather(idxbuf, [i50 + (h * 16 * S + j)])
                    ind[j, pl.ds(h * 16, 16)] = v
            for b in range(KB):
                fire_gather(b, b)

            def jg_body(g, c2):
                j0 = g * KB
                for b in range(KB):
                    j = j0 + b
                    wait_gather(b)

                    @pl.when(jnp.logical_or(g > 0, btl > 0))
                    def _():
                        wait_writes(b)

                    src = rowb.at[b]
                    dst = trb.at[b]

                    @plsc.parallel_loop(0, D, unroll=4)
                    def _(dd):
                        col = jnp.broadcast_to(dd, (16,))
                        for l0 in range(0, 128, 16):
                            v = plsc.load_gather(src, [it + l0, col])
                            dst[dd // 8, dd % 8, pl.ds(l0, 16)] = v

                    fire_writes(j, btg, b)

                    @pl.when(g < S // KB - 1)
                    def _():
                        fire_gather(j + KB, b)

                return c2

            lax.fori_loop(0, S // KB, jg_body, 0)

            return carry

        lax.fori_loop(0, TPW, bt_body, 0)
        for b in range(KB):
            wait_writes(b)

    return k


def kernel(emojis, table):
    NB, S = emojis.shape
    V, D = table.shape
    idx = emojis.reshape(-1)
    table_t = table.T
    blocks = _make_rearrange(V, D)(table_t)
    tail = table[_FULL * 128 :, :]
    table_lin = _make_transpose(V, D)(blocks, tail)
    out5 = _make_gather(V, D, NB, S)(table_lin, idx)
    out = out5.transpose(2, 4, 0, 1, 3).reshape(NB, S, D)
    return out
